# baseline (speedup 1.0000x reference)
import functools

import jax
import jax.numpy as jnp
from jax import lax
from jax.experimental import pallas as pl
from jax.experimental.pallas import tpu as pltpu
from jax.experimental.pallas import tpu_sc as plsc


def _matmuls_body(x_ref, w0_ref, w1_ref, ws_ref, b_ref,
                  h0_ref, h1_ref, s_ref):
    x = x_ref[...]

    def padded(w):
        h = jnp.dot(x, w, preferred_element_type=jnp.float32)
        hp = jnp.concatenate([h, jnp.zeros((h.shape[0], 16), jnp.float32)],
                             axis=1)
        col = lax.broadcasted_iota(jnp.int32, hp.shape, 1)
        return jnp.where(col == 128, 1.0, hp)

    h0_ref[...] = padded(w0_ref[...])
    h1_ref[...] = padded(w1_ref[...])
    s_ref[...] = (jnp.dot(x, ws_ref[...], preferred_element_type=jnp.float32)
                  + b_ref[...])


def _finalize_body(s_ref, a0_ref, a1_ref, o_ref):
    a0 = a0_ref[...]
    a1 = a1_ref[...]
    d0 = jnp.maximum(a0[:, 128:129], 1.0)
    d1 = jnp.maximum(a1[:, 128:129], 1.0)
    acc = a0[:, :128] / d0 + a1[:, :128] / d1 + s_ref[...]
    o_ref[...] = jnp.maximum(acc, 0.0)


def _make_sc_kernel(NP, DP, NCH, CH, R_T):
    mesh = plsc.VectorSubcoreMesh(core_axis_name="c", subcore_axis_name="s")

    @functools.partial(
        pl.kernel,
        out_type=(
            jax.ShapeDtypeStruct((NP, DP), jnp.float32),
            jax.ShapeDtypeStruct((NP, DP), jnp.float32),
        ),
        mesh=mesh,
        compiler_params=pltpu.CompilerParams(use_tc_tiling_on_sc=False),
        scratch_types=[
            pltpu.VMEM((2, CH), jnp.int32),
            pltpu.VMEM((2, CH), jnp.int32),
            pltpu.VMEM((CH, DP), jnp.float32),
            pltpu.VMEM((CH, DP), jnp.float32),
            pltpu.VMEM_SHARED((NP, DP), jnp.float32),
            pltpu.SemaphoreType.DMA,
            pltpu.SemaphoreType.DMA,
            pltpu.SemaphoreType.DMA,
            pltpu.SemaphoreType.DMA,
        ],
    )
    def sc_kernel(h0, h1, e0, e1, zrows, out0, out1,
                  idx_a, idx_b, rows_a, rows_b, agg_sh,
                  gsem_a, gsem_b, isem_a, isem_b):
        c = lax.axis_index("c")
        s = lax.axis_index("s")

        def process(h_hbm, edge_hbm, out_hbm):
            rbase = s * R_T

            def idx_start(j, idx_v, sem):
                pltpu.async_copy(edge_hbm.at[s * NCH + j], idx_v, sem)

            def idx_wait(idx_v, sem):
                pltpu.make_async_copy(edge_hbm.at[0], idx_v, sem).wait()

            def gather_start(idx_v, rows_v, sem):
                pltpu.async_copy(h_hbm.at[idx_v.at[0]], rows_v, sem)

            def gather_wait(rows_v, sem):
                pltpu.make_async_copy(h_hbm.at[idx_a.at[0]], rows_v,
                                      sem).wait()

            def scatter(rows_v, idx_v):
                pltpu.sync_copy(rows_v, agg_sh.at[idx_v.at[1]], add=True)

            idx_start(0, idx_a, isem_a)
            idx_start(1, idx_b, isem_b)
            pltpu.sync_copy(zrows.at[pl.ds(rbase, R_T)],
                            agg_sh.at[pl.ds(rbase, R_T)])
            plsc.subcore_barrier()

            idx_wait(idx_a, isem_a)
            gather_start(idx_a, rows_a, gsem_a)

            def pair(g, carry):
                j = 2 * g

                @pl.when(j + 1 < NCH)
                def _():
                    idx_wait(idx_b, isem_b)
                    gather_start(idx_b, rows_b, gsem_b)

                gather_wait(rows_a, gsem_a)
                scatter(rows_a, idx_a)

                @pl.when(j + 2 < NCH)
                def _():
                    idx_start(j + 2, idx_a, isem_a)
                    idx_wait(idx_a, isem_a)
                    gather_start(idx_a, rows_a, gsem_a)

                @pl.when(j + 1 < NCH)
                def _():
                    gather_wait(rows_b, gsem_b)
                    scatter(rows_b, idx_b)

                @pl.when(j + 3 < NCH)
                def _():
                    idx_start(j + 3, idx_b, isem_b)

                return carry

            lax.fori_loop(0, (NCH + 1) // 2, pair, 0)
            plsc.subcore_barrier()
            pltpu.sync_copy(agg_sh.at[pl.ds(rbase, R_T)],
                            out_hbm.at[pl.ds(rbase, R_T)])

        @pl.when(c == 0)
        def _():
            process(h0, e0, out0)

        @pl.when(c == 1)
        def _():
            process(h1, e1, out1)

    return sc_kernel


def kernel(node_features, src_rel0, dst_rel0, src_rel1, dst_rel1,
           W_rel0, W_rel1, W_self, b_self):
    N, D = node_features.shape
    D_OUT = W_rel0.shape[1]
    E = src_rel0.shape[0]
    DP = D_OUT + 16
    BM = 400
    NB = N // BM
    NT = 16
    E_T = E // NT
    CH = 128
    NCH = -(-E_T // CH)
    E_TP = NCH * CH
    NP = 10240
    R_T = NP // NT

    b2 = b_self.reshape(1, D_OUT)
    H0, H1, S = pl.pallas_call(
        _matmuls_body,
        grid=(NB,),
        in_specs=[
            pl.BlockSpec((BM, D), lambda i: (i, 0)),
            pl.BlockSpec((D, D_OUT), lambda i: (0, 0)),
            pl.BlockSpec((D, D_OUT), lambda i: (0, 0)),
            pl.BlockSpec((D, D_OUT), lambda i: (0, 0)),
            pl.BlockSpec((1, D_OUT), lambda i: (0, 0)),
        ],
        out_specs=[
            pl.BlockSpec((BM, DP), lambda i: (i, 0)),
            pl.BlockSpec((BM, DP), lambda i: (i, 0)),
            pl.BlockSpec((BM, D_OUT), lambda i: (i, 0)),
        ],
        out_shape=[
            jax.ShapeDtypeStruct((N, DP), jnp.float32),
            jax.ShapeDtypeStruct((N, DP), jnp.float32),
            jax.ShapeDtypeStruct((N, D_OUT), jnp.float32),
        ],
    )(node_features, W_rel0, W_rel1, W_self, b2)

    def pack_edges(src_idx, dst_idx):
        sp = jnp.pad(src_idx.reshape(NT, E_T), ((0, 0), (0, E_TP - E_T)))
        dp = jnp.pad(dst_idx.reshape(NT, E_T), ((0, 0), (0, E_TP - E_T)),
                     constant_values=NP - 1)
        return jnp.stack(
            [sp.reshape(NT, NCH, CH), dp.reshape(NT, NCH, CH)], axis=2
        ).reshape(NT * NCH, 2, CH)

    zrows = jnp.zeros((NP, DP), jnp.float32)
    sc_fn = _make_sc_kernel(NP, DP, NCH, CH, R_T)
    agg0, agg1 = sc_fn(H0, H1, pack_edges(src_rel0, dst_rel0),
                       pack_edges(src_rel1, dst_rel1), zrows)

    out = pl.pallas_call(
        _finalize_body,
        grid=(NB,),
        in_specs=[
            pl.BlockSpec((BM, D_OUT), lambda i: (i, 0)),
            pl.BlockSpec((BM, DP), lambda i: (i, 0)),
            pl.BlockSpec((BM, DP), lambda i: (i, 0)),
        ],
        out_specs=pl.BlockSpec((BM, D_OUT), lambda i: (i, 0)),
        out_shape=jax.ShapeDtypeStruct((N, D_OUT), jnp.float32),
    )(S, agg0, agg1)
    return out


# speedup vs baseline: 1.0575x; 1.0575x over previous
import functools

import jax
import jax.numpy as jnp
from jax import lax
from jax.experimental import pallas as pl
from jax.experimental.pallas import tpu as pltpu
from jax.experimental.pallas import tpu_sc as plsc


def _matmuls_body(x_ref, w0_ref, w1_ref, ws_ref, b_ref,
                  h0_ref, h1_ref, s_ref):
    x = x_ref[...]

    def padded(w):
        h = jnp.dot(x, w, preferred_element_type=jnp.float32)
        hp = jnp.concatenate([h, jnp.zeros((h.shape[0], 16), jnp.float32)],
                             axis=1)
        col = lax.broadcasted_iota(jnp.int32, hp.shape, 1)
        return jnp.where(col == 128, 1.0, hp)

    h0_ref[...] = padded(w0_ref[...])
    h1_ref[...] = padded(w1_ref[...])
    s_ref[...] = (jnp.dot(x, ws_ref[...], preferred_element_type=jnp.float32)
                  + b_ref[...])


def _finalize_body(s_ref, a0_ref, a1_ref, o_ref):
    a0 = a0_ref[...]
    a1 = a1_ref[...]
    d0 = jnp.maximum(a0[:, 128:129], 1.0)
    d1 = jnp.maximum(a1[:, 128:129], 1.0)
    acc = a0[:, :128] / d0 + a1[:, :128] / d1 + s_ref[...]
    o_ref[...] = jnp.maximum(acc, 0.0)


def _make_sc_kernel(NP, DP, NCH, CH, R_T):
    mesh = plsc.VectorSubcoreMesh(core_axis_name="c", subcore_axis_name="s")

    @functools.partial(
        pl.kernel,
        out_type=(
            jax.ShapeDtypeStruct((NP, DP), jnp.float32),
            jax.ShapeDtypeStruct((NP, DP), jnp.float32),
        ),
        mesh=mesh,
        compiler_params=pltpu.CompilerParams(use_tc_tiling_on_sc=False),
        scratch_types=[
            pltpu.VMEM((2, CH), jnp.int32),
            pltpu.VMEM((2, CH), jnp.int32),
            pltpu.VMEM((2, CH), jnp.int32),
            pltpu.VMEM((CH, DP), jnp.float32),
            pltpu.VMEM((CH, DP), jnp.float32),
            pltpu.VMEM_SHARED((NP, DP), jnp.float32),
            pltpu.SemaphoreType.DMA,
            pltpu.SemaphoreType.DMA,
            pltpu.SemaphoreType.DMA,
            pltpu.SemaphoreType.DMA,
            pltpu.SemaphoreType.DMA,
        ],
    )
    def sc_kernel(h0, h1, e0, e1, zrows, out0, out1,
                  idx_a, idx_b, idx_c, rows_a, rows_b, agg_sh,
                  gsem_a, gsem_b, isem_a, isem_b, isem_c):
        c = lax.axis_index("c")
        s = lax.axis_index("s")

        def process(h_hbm, edge_hbm, out_hbm):
            rbase = s * R_T
            idx_bufs = [idx_a, idx_b, idx_c]
            isems = [isem_a, isem_b, isem_c]
            rows_bufs = [rows_a, rows_b]
            gsems = [gsem_a, gsem_b]

            def idx_start(j, t3):
                pltpu.async_copy(edge_hbm.at[s * NCH + j], idx_bufs[t3],
                                 isems[t3])

            def idx_wait(t3):
                pltpu.make_async_copy(edge_hbm.at[0], idx_bufs[t3],
                                      isems[t3]).wait()

            def gather_start(t3, t2):
                pltpu.async_copy(h_hbm.at[idx_bufs[t3].at[0]], rows_bufs[t2],
                                 gsems[t2])

            def gather_wait(t2):
                pltpu.make_async_copy(h_hbm.at[idx_a.at[0]], rows_bufs[t2],
                                      gsems[t2]).wait()

            def scatter(t2, t3):
                pltpu.sync_copy(rows_bufs[t2], agg_sh.at[idx_bufs[t3].at[1]],
                                add=True)

            idx_start(0, 0)
            idx_start(1, 1)
            idx_start(2, 2)
            pltpu.sync_copy(zrows.at[pl.ds(rbase, R_T)],
                            agg_sh.at[pl.ds(rbase, R_T)])
            plsc.subcore_barrier()

            idx_wait(0)
            gather_start(0, 0)

            def sixpack(g, carry):
                j = 6 * g
                for t in range(6):
                    k = j + t

                    @pl.when(k + 1 < NCH)
                    def _(t=t, k=k):
                        idx_wait((t + 1) % 3)
                        gather_start((t + 1) % 3, (t + 1) % 2)

                    @pl.when(k < NCH)
                    def _(t=t, k=k):
                        gather_wait(t % 2)
                        scatter(t % 2, t % 3)

                    @pl.when(k + 3 < NCH)
                    def _(t=t, k=k):
                        idx_start(k + 3, t % 3)

                return carry

            lax.fori_loop(0, (NCH + 5) // 6, sixpack, 0)
            plsc.subcore_barrier()
            pltpu.sync_copy(agg_sh.at[pl.ds(rbase, R_T)],
                            out_hbm.at[pl.ds(rbase, R_T)])

        @pl.when(c == 0)
        def _():
            process(h0, e0, out0)

        @pl.when(c == 1)
        def _():
            process(h1, e1, out1)

    return sc_kernel


def kernel(node_features, src_rel0, dst_rel0, src_rel1, dst_rel1,
           W_rel0, W_rel1, W_self, b_self):
    N, D = node_features.shape
    D_OUT = W_rel0.shape[1]
    E = src_rel0.shape[0]
    DP = D_OUT + 16
    BM = 400
    NB = N // BM
    NT = 16
    E_T = E // NT
    CH = 128
    NCH = -(-E_T // CH)
    E_TP = NCH * CH
    NP = 10240
    R_T = NP // NT

    b2 = b_self.reshape(1, D_OUT)
    H0, H1, S = pl.pallas_call(
        _matmuls_body,
        grid=(NB,),
        in_specs=[
            pl.BlockSpec((BM, D), lambda i: (i, 0)),
            pl.BlockSpec((D, D_OUT), lambda i: (0, 0)),
            pl.BlockSpec((D, D_OUT), lambda i: (0, 0)),
            pl.BlockSpec((D, D_OUT), lambda i: (0, 0)),
            pl.BlockSpec((1, D_OUT), lambda i: (0, 0)),
        ],
        out_specs=[
            pl.BlockSpec((BM, DP), lambda i: (i, 0)),
            pl.BlockSpec((BM, DP), lambda i: (i, 0)),
            pl.BlockSpec((BM, D_OUT), lambda i: (i, 0)),
        ],
        out_shape=[
            jax.ShapeDtypeStruct((N, DP), jnp.float32),
            jax.ShapeDtypeStruct((N, DP), jnp.float32),
            jax.ShapeDtypeStruct((N, D_OUT), jnp.float32),
        ],
    )(node_features, W_rel0, W_rel1, W_self, b2)

    def pack_edges(src_idx, dst_idx):
        sp = jnp.pad(src_idx.reshape(NT, E_T), ((0, 0), (0, E_TP - E_T)))
        dp = jnp.pad(dst_idx.reshape(NT, E_T), ((0, 0), (0, E_TP - E_T)),
                     constant_values=NP - 1)
        return jnp.stack(
            [sp.reshape(NT, NCH, CH), dp.reshape(NT, NCH, CH)], axis=2
        ).reshape(NT * NCH, 2, CH)

    zrows = jnp.zeros((NP, DP), jnp.float32)
    sc_fn = _make_sc_kernel(NP, DP, NCH, CH, R_T)
    agg0, agg1 = sc_fn(H0, H1, pack_edges(src_rel0, dst_rel0),
                       pack_edges(src_rel1, dst_rel1), zrows)

    out = pl.pallas_call(
        _finalize_body,
        grid=(NB,),
        in_specs=[
            pl.BlockSpec((BM, D_OUT), lambda i: (i, 0)),
            pl.BlockSpec((BM, DP), lambda i: (i, 0)),
            pl.BlockSpec((BM, DP), lambda i: (i, 0)),
        ],
        out_specs=pl.BlockSpec((BM, D_OUT), lambda i: (i, 0)),
        out_shape=jax.ShapeDtypeStruct((N, D_OUT), jnp.float32),
    )(S, agg0, agg1)
    return out


# speedup vs baseline: 1.6321x; 1.5433x over previous
import functools

import jax
import jax.numpy as jnp
from jax import lax
from jax.experimental import pallas as pl
from jax.experimental.pallas import tpu as pltpu
from jax.experimental.pallas import tpu_sc as plsc


def _matmuls_body(x_ref, w0_ref, w1_ref, ws_ref, b_ref,
                  h0_ref, h1_ref, s_ref):
    x = x_ref[...]
    h0_ref[...] = jnp.dot(x, w0_ref[...], preferred_element_type=jnp.float32)
    h1_ref[...] = jnp.dot(x, w1_ref[...], preferred_element_type=jnp.float32)
    s_ref[...] = (jnp.dot(x, ws_ref[...], preferred_element_type=jnp.float32)
                  + b_ref[...])


def _finalize_body(s_ref, a0_ref, a1_ref, d0_ref, d1_ref, o_ref):
    d0 = jnp.maximum(d0_ref[...][:, 0:1], 1.0)
    d1 = jnp.maximum(d1_ref[...][:, 0:1], 1.0)
    acc = a0_ref[...] / d0 + a1_ref[...] / d1 + s_ref[...]
    o_ref[...] = jnp.maximum(acc, 0.0)


def _make_sc_kernel(NP, D, DG, E_T, NCH, CH, R_T):
    mesh = plsc.VectorSubcoreMesh(core_axis_name="c", subcore_axis_name="s")

    @functools.partial(
        pl.kernel,
        out_type=(
            jax.ShapeDtypeStruct((NP, D), jnp.float32),
            jax.ShapeDtypeStruct((NP, D), jnp.float32),
            jax.ShapeDtypeStruct((NP, DG), jnp.float32),
            jax.ShapeDtypeStruct((NP, DG), jnp.float32),
        ),
        mesh=mesh,
        compiler_params=pltpu.CompilerParams(use_tc_tiling_on_sc=False),
        scratch_types=[
            pltpu.VMEM((NCH * CH,), jnp.int32),
            pltpu.VMEM((CH,), jnp.int32),
            pltpu.VMEM((CH,), jnp.int32),
            pltpu.VMEM((CH, D), jnp.float32),
            pltpu.VMEM((CH, D), jnp.float32),
            pltpu.VMEM((CH, DG), jnp.float32),
            pltpu.VMEM_SHARED((NP, D), jnp.float32),
            pltpu.VMEM_SHARED((NP, DG), jnp.float32),
            pltpu.SemaphoreType.DMA,
            pltpu.SemaphoreType.DMA,
            pltpu.SemaphoreType.DMA,
            pltpu.SemaphoreType.DMA,
            pltpu.SemaphoreType.DMA,
            pltpu.SemaphoreType.DMA,
        ],
    )
    def sc_kernel(h0, h1, s0, d0, s1, d1, zrows, ones, out0, out1,
                  deg0, deg1,
                  src_v, dst_a, dst_b, rows_a, rows_b, ones_v,
                  agg_sh, deg_sh,
                  gsem_a, gsem_b, dsem_a, dsem_b, esem_a, esem_b):
        c = lax.axis_index("c")
        s = lax.axis_index("s")

        def process(h_hbm, src_hbm, dst_hbm, out_hbm, deg_hbm):
            rbase = s * R_T
            ebase = s * E_T
            pltpu.async_copy(src_hbm.at[pl.ds(ebase, E_T)], src_v, gsem_a)
            pltpu.async_copy(ones, ones_v, gsem_b)
            pltpu.sync_copy(zrows.at[pl.ds(rbase, R_T)],
                            agg_sh.at[pl.ds(rbase, R_T)])
            pltpu.sync_copy(zrows.at[pl.ds(rbase, R_T), pl.ds(0, DG)],
                            deg_sh.at[pl.ds(rbase, R_T)])
            pltpu.make_async_copy(src_hbm.at[pl.ds(ebase, E_T)], src_v,
                                  gsem_a).wait()
            pltpu.make_async_copy(ones, ones_v, gsem_b).wait()
            plsc.subcore_barrier()

            def gather_start(j, rows_v, sem):
                pltpu.async_copy(h_hbm.at[src_v.at[pl.ds(j * CH, CH)]],
                                 rows_v, sem)

            def gather_wait(rows_v, sem):
                pltpu.make_async_copy(h_hbm.at[src_v.at[pl.ds(0, CH)]],
                                      rows_v, sem).wait()

            def dst_start(j, dst_v, sem):
                pltpu.async_copy(dst_hbm.at[pl.ds(ebase + j * CH, CH)],
                                 dst_v, sem)

            def dst_wait(dst_v, sem):
                pltpu.make_async_copy(dst_hbm.at[pl.ds(ebase, CH)], dst_v,
                                      sem).wait()

            def deg_start(dst_v, sem):
                pltpu.async_copy(ones_v, deg_sh.at[dst_v], sem, add=True)

            def deg_wait(dst_v, sem):
                pltpu.make_async_copy(ones_v, deg_sh.at[dst_v], sem).wait()

            def scatter(rows_v, dst_v):
                pltpu.sync_copy(rows_v, agg_sh.at[dst_v], add=True)

            gather_start(0, rows_a, gsem_a)
            dst_start(0, dst_a, dsem_a)

            def pair(g, carry):
                j = 2 * g

                @pl.when(j + 1 < NCH)
                def _():
                    gather_start(j + 1, rows_b, gsem_b)
                    dst_start(j + 1, dst_b, dsem_b)

                gather_wait(rows_a, gsem_a)
                dst_wait(dst_a, dsem_a)
                deg_start(dst_a, esem_a)
                scatter(rows_a, dst_a)

                @pl.when(j + 2 < NCH)
                def _():
                    deg_wait(dst_a, esem_a)
                    gather_start(j + 2, rows_a, gsem_a)
                    dst_start(j + 2, dst_a, dsem_a)

                @pl.when(j + 1 < NCH)
                def _():
                    gather_wait(rows_b, gsem_b)
                    dst_wait(dst_b, dsem_b)
                    deg_start(dst_b, esem_b)
                    scatter(rows_b, dst_b)

                @pl.when(j + 3 < NCH)
                def _():
                    deg_wait(dst_b, esem_b)

                return carry

            lax.fori_loop(0, (NCH + 1) // 2, pair, 0)
            deg_wait(dst_a, esem_a)

            @pl.when(NCH > 1)
            def _():
                deg_wait(dst_b, esem_b)

            plsc.subcore_barrier()
            pltpu.sync_copy(agg_sh.at[pl.ds(rbase, R_T)],
                            out_hbm.at[pl.ds(rbase, R_T)])
            pltpu.sync_copy(deg_sh.at[pl.ds(rbase, R_T)],
                            deg_hbm.at[pl.ds(rbase, R_T)])

        @pl.when(c == 0)
        def _():
            process(h0, s0, d0, out0, deg0)

        @pl.when(c == 1)
        def _():
            process(h1, s1, d1, out1, deg1)

    return sc_kernel


def kernel(node_features, src_rel0, dst_rel0, src_rel1, dst_rel1,
           W_rel0, W_rel1, W_self, b_self):
    N, D = node_features.shape
    D_OUT = W_rel0.shape[1]
    E = src_rel0.shape[0]
    DG = 16
    BM = 400
    NB = N // BM
    NT = 16
    E_T = E // NT
    CH = 80
    NCH = E_T // CH
    NP = 10240
    R_T = NP // NT

    b2 = b_self.reshape(1, D_OUT)
    H0, H1, S = pl.pallas_call(
        _matmuls_body,
        grid=(NB,),
        in_specs=[
            pl.BlockSpec((BM, D), lambda i: (i, 0)),
            pl.BlockSpec((D, D_OUT), lambda i: (0, 0)),
            pl.BlockSpec((D, D_OUT), lambda i: (0, 0)),
            pl.BlockSpec((D, D_OUT), lambda i: (0, 0)),
            pl.BlockSpec((1, D_OUT), lambda i: (0, 0)),
        ],
        out_specs=[
            pl.BlockSpec((BM, D_OUT), lambda i: (i, 0)),
            pl.BlockSpec((BM, D_OUT), lambda i: (i, 0)),
            pl.BlockSpec((BM, D_OUT), lambda i: (i, 0)),
        ],
        out_shape=[
            jax.ShapeDtypeStruct((N, D_OUT), jnp.float32),
            jax.ShapeDtypeStruct((N, D_OUT), jnp.float32),
            jax.ShapeDtypeStruct((N, D_OUT), jnp.float32),
        ],
    )(node_features, W_rel0, W_rel1, W_self, b2)

    zrows = jnp.zeros((NP, D_OUT), jnp.float32)
    ones = jnp.ones((CH, DG), jnp.float32)
    sc_fn = _make_sc_kernel(NP, D_OUT, DG, E_T, NCH, CH, R_T)
    agg0, agg1, deg0, deg1 = sc_fn(H0, H1, src_rel0, dst_rel0,
                                   src_rel1, dst_rel1, zrows, ones)

    out = pl.pallas_call(
        _finalize_body,
        grid=(NB,),
        in_specs=[
            pl.BlockSpec((BM, D_OUT), lambda i: (i, 0)),
            pl.BlockSpec((BM, D_OUT), lambda i: (i, 0)),
            pl.BlockSpec((BM, D_OUT), lambda i: (i, 0)),
            pl.BlockSpec((BM, DG), lambda i: (i, 0)),
            pl.BlockSpec((BM, DG), lambda i: (i, 0)),
        ],
        out_specs=pl.BlockSpec((BM, D_OUT), lambda i: (i, 0)),
        out_shape=jax.ShapeDtypeStruct((N, D_OUT), jnp.float32),
    )(S, agg0, agg1, deg0, deg1)
    return out


# speedup vs baseline: 1.7599x; 1.0783x over previous
import functools

import jax
import jax.numpy as jnp
from jax import lax
from jax.experimental import pallas as pl
from jax.experimental.pallas import tpu as pltpu
from jax.experimental.pallas import tpu_sc as plsc


def _matmuls_body(x_ref, w0_ref, w1_ref, ws_ref, b_ref,
                  h0_ref, h1_ref, s_ref):
    x = x_ref[...]
    h0_ref[...] = jnp.dot(x, w0_ref[...], preferred_element_type=jnp.float32)
    h1_ref[...] = jnp.dot(x, w1_ref[...], preferred_element_type=jnp.float32)
    s_ref[...] = (jnp.dot(x, ws_ref[...], preferred_element_type=jnp.float32)
                  + b_ref[...])


def _finalize_body(s_ref, a0_ref, a1_ref, d0_ref, d1_ref, o_ref):
    r0 = 1.0 / jnp.maximum(d0_ref[...][:, 0:1], 1.0)
    r1 = 1.0 / jnp.maximum(d1_ref[...][:, 0:1], 1.0)
    acc = a0_ref[...] * r0 + a1_ref[...] * r1 + s_ref[...]
    o_ref[...] = jnp.maximum(acc, 0.0)


def _make_sc_kernel(NP, D, DG, E_T, NCH, CH, R_T):
    mesh = plsc.VectorSubcoreMesh(core_axis_name="c", subcore_axis_name="s")

    @functools.partial(
        pl.kernel,
        out_type=(
            jax.ShapeDtypeStruct((NP, D), jnp.float32),
            jax.ShapeDtypeStruct((NP, D), jnp.float32),
            jax.ShapeDtypeStruct((NP, DG), jnp.float32),
            jax.ShapeDtypeStruct((NP, DG), jnp.float32),
        ),
        mesh=mesh,
        compiler_params=pltpu.CompilerParams(use_tc_tiling_on_sc=False),
        scratch_types=[
            pltpu.VMEM((NCH * CH,), jnp.int32),
            pltpu.VMEM((CH,), jnp.int32),
            pltpu.VMEM((CH,), jnp.int32),
            pltpu.VMEM((CH, D), jnp.float32),
            pltpu.VMEM((CH, D), jnp.float32),
            pltpu.VMEM((CH, DG), jnp.float32),
            pltpu.VMEM_SHARED((NP, D), jnp.float32),
            pltpu.VMEM_SHARED((NP, DG), jnp.float32),
            pltpu.SemaphoreType.DMA,
            pltpu.SemaphoreType.DMA,
            pltpu.SemaphoreType.DMA,
            pltpu.SemaphoreType.DMA,
            pltpu.SemaphoreType.DMA,
            pltpu.SemaphoreType.DMA,
        ],
    )
    def sc_kernel(h0, h1, s0, d0, s1, d1, zrows, ones, out0, out1,
                  deg0, deg1,
                  src_v, dst_a, dst_b, rows_a, rows_b, ones_v,
                  agg_sh, deg_sh,
                  gsem_a, gsem_b, dsem_a, dsem_b, esem_a, esem_b):
        c = lax.axis_index("c")
        s = lax.axis_index("s")

        def process(h_hbm, src_hbm, dst_hbm, out_hbm, deg_hbm):
            rbase = s * R_T
            ebase = s * E_T
            pltpu.async_copy(src_hbm.at[pl.ds(ebase, E_T)], src_v, gsem_a)
            pltpu.async_copy(ones, ones_v, gsem_b)
            pltpu.sync_copy(zrows.at[pl.ds(rbase, R_T)],
                            agg_sh.at[pl.ds(rbase, R_T)])
            pltpu.sync_copy(zrows.at[pl.ds(rbase, R_T), pl.ds(0, DG)],
                            deg_sh.at[pl.ds(rbase, R_T)])
            pltpu.make_async_copy(src_hbm.at[pl.ds(ebase, E_T)], src_v,
                                  gsem_a).wait()
            pltpu.make_async_copy(ones, ones_v, gsem_b).wait()
            plsc.subcore_barrier()

            def gather_start(j, rows_v, sem):
                pltpu.async_copy(h_hbm.at[src_v.at[pl.ds(j * CH, CH)]],
                                 rows_v, sem)

            def gather_wait(rows_v, sem):
                pltpu.make_async_copy(h_hbm.at[src_v.at[pl.ds(0, CH)]],
                                      rows_v, sem).wait()

            def dst_start(j, dst_v, sem):
                pltpu.async_copy(dst_hbm.at[pl.ds(ebase + j * CH, CH)],
                                 dst_v, sem)

            def dst_wait(dst_v, sem):
                pltpu.make_async_copy(dst_hbm.at[pl.ds(ebase, CH)], dst_v,
                                      sem).wait()

            def deg_start(dst_v, sem):
                pltpu.async_copy(ones_v, deg_sh.at[dst_v], sem, add=True)

            def deg_wait(dst_v, sem):
                pltpu.make_async_copy(ones_v, deg_sh.at[dst_v], sem).wait()

            def scatter(rows_v, dst_v):
                pltpu.sync_copy(rows_v, agg_sh.at[dst_v], add=True)

            gather_start(0, rows_a, gsem_a)
            dst_start(0, dst_a, dsem_a)

            def pair(g, carry):
                j = 2 * g

                @pl.when(j + 1 < NCH)
                def _():
                    gather_start(j + 1, rows_b, gsem_b)
                    dst_start(j + 1, dst_b, dsem_b)

                gather_wait(rows_a, gsem_a)
                dst_wait(dst_a, dsem_a)
                deg_start(dst_a, esem_a)
                scatter(rows_a, dst_a)

                @pl.when(j + 2 < NCH)
                def _():
                    deg_wait(dst_a, esem_a)
                    gather_start(j + 2, rows_a, gsem_a)
                    dst_start(j + 2, dst_a, dsem_a)

                @pl.when(j + 1 < NCH)
                def _():
                    gather_wait(rows_b, gsem_b)
                    dst_wait(dst_b, dsem_b)
                    deg_start(dst_b, esem_b)
                    scatter(rows_b, dst_b)

                @pl.when(j + 3 < NCH)
                def _():
                    deg_wait(dst_b, esem_b)

                return carry

            lax.fori_loop(0, (NCH + 1) // 2, pair, 0)
            deg_wait(dst_a, esem_a)

            @pl.when(NCH > 1)
            def _():
                deg_wait(dst_b, esem_b)

            plsc.subcore_barrier()
            pltpu.sync_copy(agg_sh.at[pl.ds(rbase, R_T)],
                            out_hbm.at[pl.ds(rbase, R_T)])
            pltpu.sync_copy(deg_sh.at[pl.ds(rbase, R_T)],
                            deg_hbm.at[pl.ds(rbase, R_T)])

        @pl.when(c == 0)
        def _():
            process(h0, s0, d0, out0, deg0)

        @pl.when(c == 1)
        def _():
            process(h1, s1, d1, out1, deg1)

    return sc_kernel


def kernel(node_features, src_rel0, dst_rel0, src_rel1, dst_rel1,
           W_rel0, W_rel1, W_self, b_self):
    N, D = node_features.shape
    D_OUT = W_rel0.shape[1]
    E = src_rel0.shape[0]
    DG = 16
    BM = 1000
    NB = N // BM
    NT = 16
    E_T = E // NT
    CH = 80
    NCH = E_T // CH
    NP = 10240
    R_T = NP // NT

    b2 = b_self.reshape(1, D_OUT)
    H0, H1, S = pl.pallas_call(
        _matmuls_body,
        grid=(NB,),
        in_specs=[
            pl.BlockSpec((BM, D), lambda i: (i, 0)),
            pl.BlockSpec((D, D_OUT), lambda i: (0, 0)),
            pl.BlockSpec((D, D_OUT), lambda i: (0, 0)),
            pl.BlockSpec((D, D_OUT), lambda i: (0, 0)),
            pl.BlockSpec((1, D_OUT), lambda i: (0, 0)),
        ],
        out_specs=[
            pl.BlockSpec((BM, D_OUT), lambda i: (i, 0)),
            pl.BlockSpec((BM, D_OUT), lambda i: (i, 0)),
            pl.BlockSpec((BM, D_OUT), lambda i: (i, 0)),
        ],
        out_shape=[
            jax.ShapeDtypeStruct((N, D_OUT), jnp.float32),
            jax.ShapeDtypeStruct((N, D_OUT), jnp.float32),
            jax.ShapeDtypeStruct((N, D_OUT), jnp.float32),
        ],
    )(node_features, W_rel0, W_rel1, W_self, b2)

    zrows = jnp.zeros((NP, D_OUT), jnp.float32)
    ones = jnp.ones((CH, DG), jnp.float32)
    sc_fn = _make_sc_kernel(NP, D_OUT, DG, E_T, NCH, CH, R_T)
    agg0, agg1, deg0, deg1 = sc_fn(H0, H1, src_rel0, dst_rel0,
                                   src_rel1, dst_rel1, zrows, ones)

    out = pl.pallas_call(
        _finalize_body,
        grid=(NB,),
        in_specs=[
            pl.BlockSpec((BM, D_OUT), lambda i: (i, 0)),
            pl.BlockSpec((BM, D_OUT), lambda i: (i, 0)),
            pl.BlockSpec((BM, D_OUT), lambda i: (i, 0)),
            pl.BlockSpec((BM, DG), lambda i: (i, 0)),
            pl.BlockSpec((BM, DG), lambda i: (i, 0)),
        ],
        out_specs=pl.BlockSpec((BM, D_OUT), lambda i: (i, 0)),
        out_shape=jax.ShapeDtypeStruct((N, D_OUT), jnp.float32),
    )(S, agg0, agg1, deg0, deg1)
    return out


# speedup vs baseline: 1.8177x; 1.0329x over previous
import functools

import jax
import jax.numpy as jnp
from jax import lax
from jax.experimental import pallas as pl
from jax.experimental.pallas import tpu as pltpu
from jax.experimental.pallas import tpu_sc as plsc


def _matmuls_body(x_ref, w0_ref, w1_ref, ws_ref, b_ref,
                  h0_ref, h1_ref, s_ref):
    x = x_ref[...]
    h0_ref[...] = jnp.dot(x, w0_ref[...], preferred_element_type=jnp.float32)
    h1_ref[...] = jnp.dot(x, w1_ref[...], preferred_element_type=jnp.float32)
    s_ref[...] = (jnp.dot(x, ws_ref[...], preferred_element_type=jnp.float32)
                  + b_ref[...])


def _finalize_body(s_ref, a0_ref, a1_ref, d0_ref, d1_ref, o_ref):
    r0 = 1.0 / jnp.maximum(d0_ref[...][:, 0:1], 1.0)
    r1 = 1.0 / jnp.maximum(d1_ref[...][:, 0:1], 1.0)
    acc = a0_ref[...] * r0 + a1_ref[...] * r1 + s_ref[...]
    o_ref[...] = jnp.maximum(acc, 0.0)


def _make_sc_kernel(NP, D, DG, E_T, NCH, CH, R_T):
    mesh = plsc.VectorSubcoreMesh(core_axis_name="c", subcore_axis_name="s")

    @functools.partial(
        pl.kernel,
        out_type=(
            jax.ShapeDtypeStruct((NP, D), jnp.float32),
            jax.ShapeDtypeStruct((NP, D), jnp.float32),
            jax.ShapeDtypeStruct((NP, DG), jnp.float32),
            jax.ShapeDtypeStruct((NP, DG), jnp.float32),
        ),
        mesh=mesh,
        compiler_params=pltpu.CompilerParams(use_tc_tiling_on_sc=False),
        scratch_types=[
            pltpu.VMEM((NCH * CH,), jnp.int32),
            pltpu.VMEM((CH,), jnp.int32),
            pltpu.VMEM((CH,), jnp.int32),
            pltpu.VMEM((CH, D), jnp.float32),
            pltpu.VMEM((CH, D), jnp.float32),
            pltpu.VMEM((CH, DG), jnp.float32),
            pltpu.VMEM_SHARED((NP, D), jnp.float32),
            pltpu.VMEM_SHARED((NP, DG), jnp.float32),
            pltpu.SemaphoreType.DMA,
            pltpu.SemaphoreType.DMA,
            pltpu.SemaphoreType.DMA,
            pltpu.SemaphoreType.DMA,
            pltpu.SemaphoreType.DMA,
            pltpu.SemaphoreType.DMA,
        ],
    )
    def sc_kernel(h0, h1, s0, d0, s1, d1, zrows, ones, out0, out1,
                  deg0, deg1,
                  src_v, dst_a, dst_b, rows_a, rows_b, ones_v,
                  agg_sh, deg_sh,
                  gsem_a, gsem_b, dsem_a, dsem_b, esem_a, esem_b):
        c = lax.axis_index("c")
        s = lax.axis_index("s")

        def process(h_hbm, src_hbm, dst_hbm, out_hbm, deg_hbm):
            rbase = s * R_T
            ebase = s * E_T
            pltpu.async_copy(src_hbm.at[pl.ds(ebase, E_T)], src_v, gsem_a)
            pltpu.async_copy(ones, ones_v, gsem_b)
            pltpu.sync_copy(zrows.at[pl.ds(rbase, R_T)],
                            agg_sh.at[pl.ds(rbase, R_T)])
            pltpu.sync_copy(zrows.at[pl.ds(rbase, R_T), pl.ds(0, DG)],
                            deg_sh.at[pl.ds(rbase, R_T)])
            pltpu.make_async_copy(src_hbm.at[pl.ds(ebase, E_T)], src_v,
                                  gsem_a).wait()
            pltpu.make_async_copy(ones, ones_v, gsem_b).wait()
            plsc.subcore_barrier()

            def gather_start(j, rows_v, sem):
                pltpu.async_copy(h_hbm.at[src_v.at[pl.ds(j * CH, CH)]],
                                 rows_v, sem)

            def gather_wait(rows_v, sem):
                pltpu.make_async_copy(h_hbm.at[src_v.at[pl.ds(0, CH)]],
                                      rows_v, sem).wait()

            def dst_start(j, dst_v, sem):
                pltpu.async_copy(dst_hbm.at[pl.ds(ebase + j * CH, CH)],
                                 dst_v, sem)

            def dst_wait(dst_v, sem):
                pltpu.make_async_copy(dst_hbm.at[pl.ds(ebase, CH)], dst_v,
                                      sem).wait()

            def deg_start(dst_v, sem):
                pltpu.async_copy(ones_v, deg_sh.at[dst_v], sem, add=True)

            def deg_wait(dst_v, sem):
                pltpu.make_async_copy(ones_v, deg_sh.at[dst_v], sem).wait()

            def scatter(rows_v, dst_v):
                pltpu.sync_copy(rows_v, agg_sh.at[dst_v], add=True)

            gather_start(0, rows_a, gsem_a)
            dst_start(0, dst_a, dsem_a)

            def pair(g, carry):
                j = 2 * g

                @pl.when(j + 1 < NCH)
                def _():
                    gather_start(j + 1, rows_b, gsem_b)
                    dst_start(j + 1, dst_b, dsem_b)

                gather_wait(rows_a, gsem_a)
                dst_wait(dst_a, dsem_a)
                deg_start(dst_a, esem_a)
                scatter(rows_a, dst_a)

                @pl.when(j + 2 < NCH)
                def _():
                    deg_wait(dst_a, esem_a)
                    gather_start(j + 2, rows_a, gsem_a)
                    dst_start(j + 2, dst_a, dsem_a)

                @pl.when(j + 1 < NCH)
                def _():
                    gather_wait(rows_b, gsem_b)
                    dst_wait(dst_b, dsem_b)
                    deg_start(dst_b, esem_b)
                    scatter(rows_b, dst_b)

                @pl.when(j + 3 < NCH)
                def _():
                    deg_wait(dst_b, esem_b)

                return carry

            lax.fori_loop(0, (NCH + 1) // 2, pair, 0)
            deg_wait(dst_a, esem_a)

            @pl.when(NCH > 1)
            def _():
                deg_wait(dst_b, esem_b)

            plsc.subcore_barrier()
            pltpu.sync_copy(agg_sh.at[pl.ds(rbase, R_T)],
                            out_hbm.at[pl.ds(rbase, R_T)])
            pltpu.sync_copy(deg_sh.at[pl.ds(rbase, R_T)],
                            deg_hbm.at[pl.ds(rbase, R_T)])

        @pl.when(c == 0)
        def _():
            process(h0, s0, d0, out0, deg0)

        @pl.when(c == 1)
        def _():
            process(h1, s1, d1, out1, deg1)

    return sc_kernel


def kernel(node_features, src_rel0, dst_rel0, src_rel1, dst_rel1,
           W_rel0, W_rel1, W_self, b_self):
    N, D = node_features.shape
    D_OUT = W_rel0.shape[1]
    E = src_rel0.shape[0]
    DG = 16
    BM = 2000
    NB = N // BM
    NT = 16
    E_T = E // NT
    CH = 80
    NCH = E_T // CH
    NP = 10240
    R_T = NP // NT

    b2 = b_self.reshape(1, D_OUT)
    H0, H1, S = pl.pallas_call(
        _matmuls_body,
        grid=(NB,),
        in_specs=[
            pl.BlockSpec((BM, D), lambda i: (i, 0)),
            pl.BlockSpec((D, D_OUT), lambda i: (0, 0)),
            pl.BlockSpec((D, D_OUT), lambda i: (0, 0)),
            pl.BlockSpec((D, D_OUT), lambda i: (0, 0)),
            pl.BlockSpec((1, D_OUT), lambda i: (0, 0)),
        ],
        out_specs=[
            pl.BlockSpec((BM, D_OUT), lambda i: (i, 0)),
            pl.BlockSpec((BM, D_OUT), lambda i: (i, 0)),
            pl.BlockSpec((BM, D_OUT), lambda i: (i, 0)),
        ],
        out_shape=[
            jax.ShapeDtypeStruct((N, D_OUT), jnp.float32),
            jax.ShapeDtypeStruct((N, D_OUT), jnp.float32),
            jax.ShapeDtypeStruct((N, D_OUT), jnp.float32),
        ],
    )(node_features, W_rel0, W_rel1, W_self, b2)

    zrows = jnp.zeros((NP, D_OUT), jnp.float32)
    ones = jnp.ones((CH, DG), jnp.float32)
    sc_fn = _make_sc_kernel(NP, D_OUT, DG, E_T, NCH, CH, R_T)
    agg0, agg1, deg0, deg1 = sc_fn(H0, H1, src_rel0, dst_rel0,
                                   src_rel1, dst_rel1, zrows, ones)

    out = pl.pallas_call(
        _finalize_body,
        grid=(NB,),
        in_specs=[
            pl.BlockSpec((BM, D_OUT), lambda i: (i, 0)),
            pl.BlockSpec((BM, D_OUT), lambda i: (i, 0)),
            pl.BlockSpec((BM, D_OUT), lambda i: (i, 0)),
            pl.BlockSpec((BM, DG), lambda i: (i, 0)),
            pl.BlockSpec((BM, DG), lambda i: (i, 0)),
        ],
        out_specs=pl.BlockSpec((BM, D_OUT), lambda i: (i, 0)),
        out_shape=jax.ShapeDtypeStruct((N, D_OUT), jnp.float32),
    )(S, agg0, agg1, deg0, deg1)
    return out
